# Initial kernel scaffold; baseline (speedup 1.0000x reference)
import jax
import jax.numpy as jnp
from jax.experimental import pallas as pl


def kernel(features, elli_points, params, edge_index):
    raise NotImplementedError("write your pallas kernel here")


# speedup vs baseline: 1.2138x; 1.2138x over previous
import functools

import jax
import jax.numpy as jnp
from jax import lax
from jax.experimental import pallas as pl
from jax.experimental.pallas import tpu as pltpu
from jax.experimental.pallas import tpu_sc as plsc

N = 10000
NPAD = 10240
FEAT = 896
RB = 256
B = 128
E_TOT = 330000
EP = 360448
NB16 = EP // (16 * B)
NH = NB16 // 2
NB32 = EP // (32 * B)
ACC = 10112
NRT = ACC // 16
SRC_PAD = N + 2
DST_DEG_PAD = N + 1


def _mesh():
    return plsc.VectorSubcoreMesh(core_axis_name="c", subcore_axis_name="s")


def _fill_zero(ref, nrows, C):
    z16 = jnp.zeros((16,), jnp.float32)

    def zrow(r, carry):
        for j in range(C // 16):
            ref[r, pl.ds(j * 16, 16)] = z16
        return carry

    lax.fori_loop(0, nrows, zrow, 0)


def _zero_acc_slice(rows, acc, base):
    for z in range(4):
        pltpu.sync_copy(rows, acc.at[pl.ds(base + z * B, B)])
    pltpu.sync_copy(rows.at[pl.ds(0, 120)], acc.at[pl.ds(base + 4 * B, 120)])


def _copy_out_slice(rows, acc, out2d, base):
    for z in range(4):
        pltpu.sync_copy(acc.at[pl.ds(base + z * B, B)], rows)
        pltpu.sync_copy(rows, out2d.at[pl.ds(base + z * B, B)])
    pltpu.sync_copy(acc.at[pl.ds(base + 4 * B, 120)], rows.at[pl.ds(0, 120)])
    pltpu.sync_copy(rows.at[pl.ds(0, 120)], out2d.at[pl.ds(base + 4 * B, 120)])


@functools.lru_cache(None)
def _make_agg_chunked(nc):
    C = 128

    @functools.partial(
        pl.kernel,
        out_type=jax.ShapeDtypeStruct((nc, NPAD, C), jnp.float32),
        mesh=_mesh(),
        scratch_types=[
            pltpu.VMEM((NH, B), jnp.int32),
            pltpu.VMEM((NH, B), jnp.int32),
            pltpu.VMEM((B, C), jnp.float32),
            pltpu.VMEM_SHARED((ACC, C), jnp.float32),
            pltpu.SemaphoreType.DMA,
        ],
    )
    def kern(h_hbm, src_hbm, dst_hbm, out_hbm, src_v, dst_v, rows, acc, sem):
        cid = lax.axis_index("c")
        sid = lax.axis_index("s")
        base = sid * NRT
        for c in range(nc):
            @pl.when(cid == (c % 2))
            def _(c=c):
                _fill_zero(rows, B, C)
                _zero_acc_slice(rows, acc, base)
                plsc.subcore_barrier()
                hc = h_hbm.at[c]
                for half in range(2):
                    pltpu.sync_copy(src_hbm.at[sid].at[half], src_v)
                    pltpu.sync_copy(dst_hbm.at[sid].at[half], dst_v)

                    def body(b, carry):
                        pltpu.async_copy(hc.at[src_v.at[b]], rows, sem).wait()
                        pltpu.sync_copy(rows, acc.at[dst_v.at[b]], add=True)
                        return carry

                    lax.fori_loop(0, NH, body, 0)
                plsc.subcore_barrier()
                _copy_out_slice(rows, acc, out_hbm.at[c], base)

    return kern


@functools.lru_cache(None)
def _make_agg_partial():
    C = 128

    @functools.partial(
        pl.kernel,
        out_type=jax.ShapeDtypeStruct((2, NPAD, C), jnp.float32),
        mesh=_mesh(),
        scratch_types=[
            pltpu.VMEM((NB32, B), jnp.int32),
            pltpu.VMEM((NB32, B), jnp.int32),
            pltpu.VMEM((B, C), jnp.float32),
            pltpu.VMEM_SHARED((ACC, C), jnp.float32),
            pltpu.SemaphoreType.DMA,
        ],
    )
    def kern(h_hbm, src_hbm, dst_hbm, out_hbm, src_v, dst_v, rows, acc, sem):
        cid = lax.axis_index("c")
        sid = lax.axis_index("s")
        wid = cid * 16 + sid
        base = sid * NRT
        pltpu.sync_copy(src_hbm.at[wid], src_v)
        pltpu.sync_copy(dst_hbm.at[wid], dst_v)
        _fill_zero(rows, B, C)
        _zero_acc_slice(rows, acc, base)
        plsc.subcore_barrier()

        def body(b, carry):
            pltpu.async_copy(h_hbm.at[src_v.at[b]], rows, sem).wait()
            pltpu.sync_copy(rows, acc.at[dst_v.at[b]], add=True)
            return carry

        lax.fori_loop(0, NB32, body, 0)
        plsc.subcore_barrier()
        _copy_out_slice(rows, acc, out_hbm.at[cid], base)

    return kern


@functools.lru_cache(None)
def _make_deg():
    C = 128

    @functools.partial(
        pl.kernel,
        out_type=jax.ShapeDtypeStruct((2, NPAD, C), jnp.float32),
        mesh=_mesh(),
        scratch_types=[
            pltpu.VMEM((NB32, B), jnp.int32),
            pltpu.VMEM((B, C), jnp.float32),
            pltpu.VMEM_SHARED((ACC, C), jnp.float32),
        ],
    )
    def kern(dst_hbm, out_hbm, dst_v, rows, acc):
        cid = lax.axis_index("c")
        sid = lax.axis_index("s")
        wid = cid * 16 + sid
        base = sid * NRT
        pltpu.sync_copy(dst_hbm.at[wid], dst_v)
        _fill_zero(rows, B, C)
        _zero_acc_slice(rows, acc, base)
        o16 = jnp.ones((16,), jnp.float32)

        def fill1(r, carry):
            for j in range(C // 16):
                rows[r, pl.ds(j * 16, 16)] = o16
            return carry

        lax.fori_loop(0, B, fill1, 0)
        plsc.subcore_barrier()

        def body(b, carry):
            pltpu.sync_copy(rows, acc.at[dst_v.at[b]], add=True)
            return carry

        lax.fori_loop(0, NB32, body, 0)
        plsc.subcore_barrier()
        _copy_out_slice(rows, acc, out_hbm.at[cid], base)

    return kern


@functools.lru_cache(None)
def _make_entry():
    nc = 8

    def body(feat_ref, tail_ref, wf_ref, wt_ref, dinv_ref, out_ref):
        h = jnp.dot(feat_ref[...], wf_ref[...], preferred_element_type=jnp.float32)
        h = h + jnp.dot(tail_ref[...], wt_ref[...], preferred_element_type=jnp.float32)
        h = h * dinv_ref[...][:, 0:1]
        for c in range(nc):
            out_ref[c] = h[:, c * 128:(c + 1) * 128]

    return pl.pallas_call(
        body,
        grid=(NPAD // RB,),
        in_specs=[
            pl.BlockSpec((RB, FEAT), lambda i: (i, 0)),
            pl.BlockSpec((RB, 128), lambda i: (i, 0)),
            pl.BlockSpec((FEAT, 1024), lambda i: (0, 0)),
            pl.BlockSpec((128, 1024), lambda i: (0, 0)),
            pl.BlockSpec((RB, 128), lambda i: (i, 0)),
        ],
        out_specs=pl.BlockSpec((nc, RB, 128), lambda i: (0, i, 0)),
        out_shape=jax.ShapeDtypeStruct((nc, NPAD, 128), jnp.float32),
    )


@functools.lru_cache(None)
def _make_conv(n_in_blk, partial_in, c_in, w_out, emit_x):
    w_in = c_in if partial_in else n_in_blk * c_in
    nc_out = max(w_out // 128, 1)
    c_out = min(w_out, 128)

    def body(s_ref, b_ref, w_ref, dinv_ref, *out_refs):
        if partial_in:
            xin = s_ref[0] + s_ref[1]
        else:
            xin = jnp.concatenate([s_ref[c] for c in range(n_in_blk)], axis=1)
        dv = dinv_ref[...][:, 0:1]
        x = jnp.maximum(dv * xin + b_ref[...], 0.0)
        h = jnp.dot(x, w_ref[...], preferred_element_type=jnp.float32) * dv
        for c in range(nc_out):
            out_refs[0][c] = h[:, c * c_out:(c + 1) * c_out]
        if emit_x:
            out_refs[1][...] = x

    out_shapes = jax.ShapeDtypeStruct((nc_out, NPAD, c_out), jnp.float32)
    out_specs = pl.BlockSpec((nc_out, RB, c_out), lambda i: (0, i, 0))
    if emit_x:
        out_shapes = [out_shapes,
                      jax.ShapeDtypeStruct((NPAD, w_in), jnp.float32)]
        out_specs = [out_specs, pl.BlockSpec((RB, w_in), lambda i: (i, 0))]
    nb = 2 if partial_in else n_in_blk
    return pl.pallas_call(
        body,
        grid=(NPAD // RB,),
        in_specs=[
            pl.BlockSpec((nb, RB, c_in), lambda i: (0, i, 0)),
            pl.BlockSpec((1, w_in), lambda i: (0, 0)),
            pl.BlockSpec((w_in, w_out), lambda i: (0, 0)),
            pl.BlockSpec((RB, 128), lambda i: (i, 0)),
        ],
        out_specs=out_specs,
        out_shape=out_shapes,
    )


@functools.lru_cache(None)
def _make_dinv():
    def body(d_ref, out_ref):
        d = d_ref[0][:, 0:1] + d_ref[1][:, 0:1]
        dv = jnp.where(d > 0, lax.rsqrt(jnp.maximum(d, 1.0)), 0.0)
        out_ref[...] = jnp.broadcast_to(dv, (RB, 128))

    return pl.pallas_call(
        body,
        grid=(NPAD // RB,),
        in_specs=[pl.BlockSpec((2, RB, 128), lambda i: (0, i, 0))],
        out_specs=pl.BlockSpec((RB, 128), lambda i: (i, 0)),
        out_shape=jax.ShapeDtypeStruct((NPAD, 128), jnp.float32),
    )


@functools.lru_cache(None)
def _make_coords():
    def body(p1_ref, p2_ref, p3_ref, b_ref, dinv_ref, out_ref):
        dv = dinv_ref[...][:, 0:1]
        for j, p in enumerate((p1_ref, p2_ref, p3_ref)):
            v = dv * (p[0] + p[1]) + b_ref[...][j:j + 1, :]
            out_ref[j] = v[:, :16]

    return pl.pallas_call(
        body,
        grid=(NPAD // RB,),
        in_specs=[
            pl.BlockSpec((2, RB, 128), lambda i: (0, i, 0)),
            pl.BlockSpec((2, RB, 128), lambda i: (0, i, 0)),
            pl.BlockSpec((2, RB, 128), lambda i: (0, i, 0)),
            pl.BlockSpec((8, 128), lambda i: (0, 0)),
            pl.BlockSpec((RB, 128), lambda i: (i, 0)),
        ],
        out_specs=pl.BlockSpec((3, RB, 16), lambda i: (0, i, 0)),
        out_shape=jax.ShapeDtypeStruct((3, NPAD, 16), jnp.float32),
    )


def _agg_chunked(h, src16, dst16, nc):
    return _make_agg_chunked(nc)(h, src16, dst16)


def _agg_partial(h, src32, dst32):
    return _make_agg_partial()(h, src32, dst32)


def kernel(features, elli_points, params, edge_index):
    f32 = jnp.float32
    ei = edge_index.astype(jnp.int32)
    loop = jnp.arange(N, dtype=jnp.int32)
    src = jnp.concatenate([ei[0], loop])
    dst = jnp.concatenate([ei[1], loop])
    pad_src = jnp.full((EP - E_TOT,), SRC_PAD, jnp.int32)
    pad_dst = jnp.zeros((EP - E_TOT,), jnp.int32)
    pad_deg = jnp.full((EP - E_TOT,), DST_DEG_PAD, jnp.int32)
    srcp = jnp.concatenate([src, pad_src])
    dstp = jnp.concatenate([dst, pad_dst])
    dstdeg = jnp.concatenate([dst, pad_deg])
    src16 = srcp.reshape(16, 2, NH, B)
    dst16 = dstp.reshape(16, 2, NH, B)
    src32 = srcp.reshape(32, NB32, B)
    dst32 = dstp.reshape(32, NB32, B)
    dst32d = dstdeg.reshape(32, NB32, B)

    featp = jnp.zeros((NPAD, FEAT), f32).at[:N].set(features)
    tail = jnp.zeros((NPAD, 128), f32).at[:N, :3].set(elli_points)

    deg = _make_deg()(dst32d)
    dinv = _make_dinv()(deg)

    cparts = []
    for bname in ("block1", "block2", "block3"):
        p = params[bname]
        Wf = p["W1"][:FEAT]
        Wt = p["W1"][FEAT:]
        if Wt.shape[0] < 128:
            Wt = jnp.zeros((128, 1024), f32).at[: Wt.shape[0]].set(Wt)
        h = _make_entry()(featp, tail, Wf, Wt, dinv)
        s = _agg_chunked(h, src16, dst16, 8)
        h = _make_conv(8, False, 128, 512, False)(
            s, p["b1"].reshape(1, 1024), p["W21"], dinv)
        s = _agg_chunked(h, src16, dst16, 4)
        h = _make_conv(4, False, 128, 256, False)(
            s, p["b21"].reshape(1, 512), p["W22"], dinv)
        s = _agg_chunked(h, src16, dst16, 2)
        h = _make_conv(2, False, 128, 128, False)(
            s, p["b22"].reshape(1, 256), p["W23"], dinv)
        s = _agg_partial(h.reshape(NPAD, 128), src32, dst32)
        W3p = jnp.zeros((128, 128), f32).at[:, :3].set(p["W3"])
        h3, tail = _make_conv(1, True, 128, 128, True)(
            s, p["b23"].reshape(1, 128), W3p, dinv)
        cparts.append(_agg_partial(h3.reshape(NPAD, 128), src32, dst32))

    b3s = jnp.zeros((8, 128), f32)
    for j, bname in enumerate(("block1", "block2", "block3")):
        b3s = b3s.at[j, :3].set(params[bname]["b3"])
    coords = _make_coords()(cparts[0], cparts[1], cparts[2], b3s, dinv)
    c1 = coords[0, :N, :3]
    c2 = coords[1, :N, :3]
    c3 = coords[2, :N, :3]
    return (elli_points, c1, c1, c2, c2, c3)


# speedup vs baseline: 1.2708x; 1.0470x over previous
import functools

import jax
import jax.numpy as jnp
from jax import lax
from jax.experimental import pallas as pl
from jax.experimental.pallas import tpu as pltpu
from jax.experimental.pallas import tpu_sc as plsc

N = 10000
NPAD = 10240
FEAT = 896
RB = 256
B = 128
E_TOT = 330000
EP = 360448
NB32 = EP // (32 * B)
BE = 32
SEC = 88
NSEC16 = EP // (16 * BE * SEC)
NSEC32 = EP // (32 * BE * SEC)
NBUF = 4
ACC = 10112
NRT = ACC // 16
SRC_PAD = N + 2
DST_DEG_PAD = N + 1


def _mesh():
    return plsc.VectorSubcoreMesh(core_axis_name="c", subcore_axis_name="s")


def _fill_zero(ref, nrows, C):
    z16 = jnp.zeros((16,), jnp.float32)

    def zrow(r, carry):
        for j in range(C // 16):
            ref[r, pl.ds(j * 16, 16)] = z16
        return carry

    lax.fori_loop(0, nrows, zrow, 0)


def _zero_acc_slice(stage, rz, acc, base):
    nf, rem = NRT // rz, NRT % rz
    for z in range(nf):
        pltpu.sync_copy(stage, acc.at[pl.ds(base + z * rz, rz)])
    if rem:
        pltpu.sync_copy(stage.at[pl.ds(0, rem)], acc.at[pl.ds(base + nf * rz, rem)])


def _copy_out_slice(stage, rz, acc, out2d, base):
    nf, rem = NRT // rz, NRT % rz
    for z in range(nf):
        pltpu.sync_copy(acc.at[pl.ds(base + z * rz, rz)], stage)
        pltpu.sync_copy(stage, out2d.at[pl.ds(base + z * rz, rz)])
    if rem:
        pltpu.sync_copy(acc.at[pl.ds(base + nf * rz, rem)], stage.at[pl.ds(0, rem)])
        pltpu.sync_copy(stage.at[pl.ds(0, rem)], out2d.at[pl.ds(base + nf * rz, rem)])


def _run_sec(hmat, src_v, dst_v, rows, sems, acc):
    for s in range(NBUF):
        pltpu.async_copy(hmat.at[src_v.at[s]], rows.at[s], sems[s])

    def group(j, carry):
        b0 = NBUF * j
        for s in range(NBUF):
            b = b0 + s
            pltpu.make_async_copy(hmat.at[src_v.at[b]], rows.at[s], sems[s]).wait()
            pltpu.sync_copy(rows.at[s], acc.at[dst_v.at[b]], add=True)
            pltpu.async_copy(hmat.at[src_v.at[b + NBUF]], rows.at[s], sems[s])
        return carry

    lax.fori_loop(0, SEC // NBUF - 1, group, 0)
    b0 = SEC - NBUF
    for s in range(NBUF):
        b = b0 + s
        pltpu.make_async_copy(hmat.at[src_v.at[b]], rows.at[s], sems[s]).wait()
        pltpu.sync_copy(rows.at[s], acc.at[dst_v.at[b]], add=True)


@functools.lru_cache(None)
def _make_agg_chunked(nc):
    C = 128

    @functools.partial(
        pl.kernel,
        out_type=jax.ShapeDtypeStruct((nc, NPAD, C), jnp.float32),
        mesh=_mesh(),
        scratch_types=[
            pltpu.VMEM((SEC, BE), jnp.int32),
            pltpu.VMEM((SEC, BE), jnp.int32),
            pltpu.VMEM((NBUF, BE, C), jnp.float32),
            pltpu.VMEM_SHARED((ACC, C), jnp.float32),
            pltpu.SemaphoreType.DMA,
            pltpu.SemaphoreType.DMA,
            pltpu.SemaphoreType.DMA,
            pltpu.SemaphoreType.DMA,
        ],
    )
    def kern(h_hbm, src_hbm, dst_hbm, out_hbm, src_v, dst_v, rows, acc,
             sem0, sem1, sem2, sem3):
        sems = (sem0, sem1, sem2, sem3)
        cid = lax.axis_index("c")
        sid = lax.axis_index("s")
        base = sid * NRT

        def chunk_body(c, carry):
            @pl.when(cid == lax.rem(c, 2))
            def _():
                _fill_zero(rows.at[0], BE, C)
                _zero_acc_slice(rows.at[0], BE, acc, base)
                plsc.subcore_barrier()

                def sec_body(sec, carry2):
                    pltpu.sync_copy(src_hbm.at[sid].at[sec], src_v)
                    pltpu.sync_copy(dst_hbm.at[sid].at[sec], dst_v)
                    _run_sec(h_hbm.at[c], src_v, dst_v, rows, sems, acc)
                    return carry2

                lax.fori_loop(0, NSEC16, sec_body, 0)
                plsc.subcore_barrier()
                _copy_out_slice(rows.at[0], BE, acc, out_hbm.at[c], base)
            return carry

        lax.fori_loop(0, nc, chunk_body, 0)

    return kern


@functools.lru_cache(None)
def _make_agg_partial():
    C = 128

    @functools.partial(
        pl.kernel,
        out_type=jax.ShapeDtypeStruct((2, NPAD, C), jnp.float32),
        mesh=_mesh(),
        scratch_types=[
            pltpu.VMEM((SEC, BE), jnp.int32),
            pltpu.VMEM((SEC, BE), jnp.int32),
            pltpu.VMEM((NBUF, BE, C), jnp.float32),
            pltpu.VMEM_SHARED((ACC, C), jnp.float32),
            pltpu.SemaphoreType.DMA,
            pltpu.SemaphoreType.DMA,
            pltpu.SemaphoreType.DMA,
            pltpu.SemaphoreType.DMA,
        ],
    )
    def kern(h_hbm, src_hbm, dst_hbm, out_hbm, src_v, dst_v, rows, acc,
             sem0, sem1, sem2, sem3):
        sems = (sem0, sem1, sem2, sem3)
        cid = lax.axis_index("c")
        sid = lax.axis_index("s")
        wid = cid * 16 + sid
        base = sid * NRT
        _fill_zero(rows.at[0], BE, C)
        _zero_acc_slice(rows.at[0], BE, acc, base)
        plsc.subcore_barrier()

        def sec_body(sec, carry2):
            pltpu.sync_copy(src_hbm.at[wid].at[sec], src_v)
            pltpu.sync_copy(dst_hbm.at[wid].at[sec], dst_v)
            _run_sec(h_hbm, src_v, dst_v, rows, sems, acc)
            return carry2

        lax.fori_loop(0, NSEC32, sec_body, 0)
        plsc.subcore_barrier()
        _copy_out_slice(rows.at[0], BE, acc, out_hbm.at[cid], base)

    return kern


@functools.lru_cache(None)
def _make_deg():
    C = 128

    @functools.partial(
        pl.kernel,
        out_type=jax.ShapeDtypeStruct((2, NPAD, C), jnp.float32),
        mesh=_mesh(),
        scratch_types=[
            pltpu.VMEM((NB32, B), jnp.int32),
            pltpu.VMEM((B, C), jnp.float32),
            pltpu.VMEM_SHARED((ACC, C), jnp.float32),
        ],
    )
    def kern(dst_hbm, out_hbm, dst_v, rows, acc):
        cid = lax.axis_index("c")
        sid = lax.axis_index("s")
        wid = cid * 16 + sid
        base = sid * NRT
        pltpu.sync_copy(dst_hbm.at[wid], dst_v)
        _fill_zero(rows, B, C)
        _zero_acc_slice(rows, B, acc, base)
        o16 = jnp.ones((16,), jnp.float32)

        def fill1(r, carry):
            for j in range(C // 16):
                rows[r, pl.ds(j * 16, 16)] = o16
            return carry

        lax.fori_loop(0, B, fill1, 0)
        plsc.subcore_barrier()

        def body(b, carry):
            pltpu.sync_copy(rows, acc.at[dst_v.at[b]], add=True)
            return carry

        lax.fori_loop(0, NB32, body, 0)
        plsc.subcore_barrier()
        _copy_out_slice(rows, B, acc, out_hbm.at[cid], base)

    return kern


@functools.lru_cache(None)
def _make_entry():
    nc = 8

    def body(feat_ref, tail_ref, wf_ref, wt_ref, dinv_ref, out_ref):
        h = jnp.dot(feat_ref[...], wf_ref[...], preferred_element_type=jnp.float32)
        h = h + jnp.dot(tail_ref[...], wt_ref[...], preferred_element_type=jnp.float32)
        h = h * dinv_ref[...][:, 0:1]
        for c in range(nc):
            out_ref[c] = h[:, c * 128:(c + 1) * 128]

    return pl.pallas_call(
        body,
        grid=(NPAD // RB,),
        in_specs=[
            pl.BlockSpec((RB, FEAT), lambda i: (i, 0)),
            pl.BlockSpec((RB, 128), lambda i: (i, 0)),
            pl.BlockSpec((FEAT, 1024), lambda i: (0, 0)),
            pl.BlockSpec((128, 1024), lambda i: (0, 0)),
            pl.BlockSpec((RB, 128), lambda i: (i, 0)),
        ],
        out_specs=pl.BlockSpec((nc, RB, 128), lambda i: (0, i, 0)),
        out_shape=jax.ShapeDtypeStruct((nc, NPAD, 128), jnp.float32),
    )


@functools.lru_cache(None)
def _make_conv(n_in_blk, partial_in, c_in, w_out, emit_x):
    w_in = c_in if partial_in else n_in_blk * c_in
    nc_out = max(w_out // 128, 1)
    c_out = min(w_out, 128)

    def body(s_ref, b_ref, w_ref, dinv_ref, *out_refs):
        if partial_in:
            xin = s_ref[0] + s_ref[1]
        else:
            xin = jnp.concatenate([s_ref[c] for c in range(n_in_blk)], axis=1)
        dv = dinv_ref[...][:, 0:1]
        x = jnp.maximum(dv * xin + b_ref[...], 0.0)
        h = jnp.dot(x, w_ref[...], preferred_element_type=jnp.float32) * dv
        for c in range(nc_out):
            out_refs[0][c] = h[:, c * c_out:(c + 1) * c_out]
        if emit_x:
            out_refs[1][...] = x

    out_shapes = jax.ShapeDtypeStruct((nc_out, NPAD, c_out), jnp.float32)
    out_specs = pl.BlockSpec((nc_out, RB, c_out), lambda i: (0, i, 0))
    if emit_x:
        out_shapes = [out_shapes,
                      jax.ShapeDtypeStruct((NPAD, w_in), jnp.float32)]
        out_specs = [out_specs, pl.BlockSpec((RB, w_in), lambda i: (i, 0))]
    nb = 2 if partial_in else n_in_blk
    return pl.pallas_call(
        body,
        grid=(NPAD // RB,),
        in_specs=[
            pl.BlockSpec((nb, RB, c_in), lambda i: (0, i, 0)),
            pl.BlockSpec((1, w_in), lambda i: (0, 0)),
            pl.BlockSpec((w_in, w_out), lambda i: (0, 0)),
            pl.BlockSpec((RB, 128), lambda i: (i, 0)),
        ],
        out_specs=out_specs,
        out_shape=out_shapes,
    )


@functools.lru_cache(None)
def _make_dinv():
    def body(d_ref, out_ref):
        d = d_ref[0][:, 0:1] + d_ref[1][:, 0:1]
        dv = jnp.where(d > 0, lax.rsqrt(jnp.maximum(d, 1.0)), 0.0)
        out_ref[...] = jnp.broadcast_to(dv, (RB, 128))

    return pl.pallas_call(
        body,
        grid=(NPAD // RB,),
        in_specs=[pl.BlockSpec((2, RB, 128), lambda i: (0, i, 0))],
        out_specs=pl.BlockSpec((RB, 128), lambda i: (i, 0)),
        out_shape=jax.ShapeDtypeStruct((NPAD, 128), jnp.float32),
    )


@functools.lru_cache(None)
def _make_coords():
    def body(p1_ref, p2_ref, p3_ref, b_ref, dinv_ref, out_ref):
        dv = dinv_ref[...][:, 0:1]
        for j, p in enumerate((p1_ref, p2_ref, p3_ref)):
            v = dv * (p[0] + p[1]) + b_ref[...][j:j + 1, :]
            out_ref[j] = v[:, :16]

    return pl.pallas_call(
        body,
        grid=(NPAD // RB,),
        in_specs=[
            pl.BlockSpec((2, RB, 128), lambda i: (0, i, 0)),
            pl.BlockSpec((2, RB, 128), lambda i: (0, i, 0)),
            pl.BlockSpec((2, RB, 128), lambda i: (0, i, 0)),
            pl.BlockSpec((8, 128), lambda i: (0, 0)),
            pl.BlockSpec((RB, 128), lambda i: (i, 0)),
        ],
        out_specs=pl.BlockSpec((3, RB, 16), lambda i: (0, i, 0)),
        out_shape=jax.ShapeDtypeStruct((3, NPAD, 16), jnp.float32),
    )


def _agg_chunked(h, src16, dst16, nc):
    return _make_agg_chunked(nc)(h, src16, dst16)


def _agg_partial(h, src32, dst32):
    return _make_agg_partial()(h, src32, dst32)


def kernel(features, elli_points, params, edge_index):
    f32 = jnp.float32
    ei = edge_index.astype(jnp.int32)
    loop = jnp.arange(N, dtype=jnp.int32)
    src = jnp.concatenate([ei[0], loop])
    dst = jnp.concatenate([ei[1], loop])
    pad_src = jnp.full((EP - E_TOT,), SRC_PAD, jnp.int32)
    pad_dst = jnp.zeros((EP - E_TOT,), jnp.int32)
    pad_deg = jnp.full((EP - E_TOT,), DST_DEG_PAD, jnp.int32)
    srcp = jnp.concatenate([src, pad_src])
    dstp = jnp.concatenate([dst, pad_dst])
    dstdeg = jnp.concatenate([dst, pad_deg])
    src16 = srcp.reshape(16, NSEC16, SEC, BE)
    dst16 = dstp.reshape(16, NSEC16, SEC, BE)
    src32 = srcp.reshape(32, NSEC32, SEC, BE)
    dst32 = dstp.reshape(32, NSEC32, SEC, BE)
    dst32d = dstdeg.reshape(32, NB32, B)

    featp = jnp.zeros((NPAD, FEAT), f32).at[:N].set(features)
    tail = jnp.zeros((NPAD, 128), f32).at[:N, :3].set(elli_points)

    deg = _make_deg()(dst32d)
    dinv = _make_dinv()(deg)

    cparts = []
    for bname in ("block1", "block2", "block3"):
        p = params[bname]
        Wf = p["W1"][:FEAT]
        Wt = p["W1"][FEAT:]
        if Wt.shape[0] < 128:
            Wt = jnp.zeros((128, 1024), f32).at[: Wt.shape[0]].set(Wt)
        h = _make_entry()(featp, tail, Wf, Wt, dinv)
        s = _agg_chunked(h, src16, dst16, 8)
        h = _make_conv(8, False, 128, 512, False)(
            s, p["b1"].reshape(1, 1024), p["W21"], dinv)
        s = _agg_chunked(h, src16, dst16, 4)
        h = _make_conv(4, False, 128, 256, False)(
            s, p["b21"].reshape(1, 512), p["W22"], dinv)
        s = _agg_chunked(h, src16, dst16, 2)
        h = _make_conv(2, False, 128, 128, False)(
            s, p["b22"].reshape(1, 256), p["W23"], dinv)
        s = _agg_partial(h.reshape(NPAD, 128), src32, dst32)
        W3p = jnp.zeros((128, 128), f32).at[:, :3].set(p["W3"])
        h3, tail = _make_conv(1, True, 128, 128, True)(
            s, p["b23"].reshape(1, 128), W3p, dinv)
        cparts.append(_agg_partial(h3.reshape(NPAD, 128), src32, dst32))

    b3s = jnp.zeros((8, 128), f32)
    for j, bname in enumerate(("block1", "block2", "block3")):
        b3s = b3s.at[j, :3].set(params[bname]["b3"])
    coords = _make_coords()(cparts[0], cparts[1], cparts[2], b3s, dinv)
    c1 = coords[0, :N, :3]
    c2 = coords[1, :N, :3]
    c3 = coords[2, :N, :3]
    return (elli_points, c1, c1, c2, c2, c3)
